# baseline (speedup 1.0000x reference)
import functools

import jax
import jax.numpy as jnp
from jax import lax
from jax.experimental import pallas as pl
from jax.experimental.pallas import tpu as pltpu
from jax.experimental.pallas import tpu_sc as plsc

_NUM_EXPERTS = 64
_TOP_K = 8
_BLOCK = 2048
_FMIN = float(jnp.finfo(jnp.float32).min)

_NC = 2
_NS = 16
_NW = _NC * _NS
_L = 16


def _gate_body(x_ref, pe_ref, wt_ref, bias_ref, nb_ref, dead_ref,
               logits_ref, probs_ref):
    x = x_ref[...]
    logits = jax.lax.dot_general(
        x, wt_ref[...], (((1,), (0,)), ((), ())),
        preferred_element_type=jnp.float32)
    pe = pe_ref[...]
    logits = logits + bias_ref[...] + pe * nb_ref[...]
    logits = jnp.where(dead_ref[...] != 0, _FMIN, logits)
    logits_ref[...] = logits
    e = jnp.exp(logits)
    s = jnp.sum(e, axis=-1, keepdims=True)
    probs_ref[...] = e * (1.0 / s)


def _gather16(src, idx):
    return lax.gather(
        src, idx[:, None],
        dimension_numbers=lax.GatherDimensionNumbers(
            offset_dims=(), collapsed_slice_dims=(0,), start_index_map=(0,)),
        slice_sizes=(1,),
        mode=lax.GatherScatterMode.PROMISE_IN_BOUNDS)


def _merge8(a, b, iota):
    b_shift = _gather16(b, (iota - 8) & 15)
    return jnp.where(iota < 8, a, b_shift)


def _sc_topk(probs_hbm, topw_hbm, topi_hbm, probs_v, topw_v, topi_v):
    rows = probs_hbm.shape[0] // (_NUM_EXPERTS * _NW)
    wid = lax.axis_index("s") * _NC + lax.axis_index("c")
    base = wid * rows * _NUM_EXPERTS
    pltpu.sync_copy(probs_hbm.at[pl.ds(base, rows * _NUM_EXPERTS)], probs_v)

    iota = lax.iota(jnp.int32, _L)

    def one_row(r):
        off = r * _NUM_EXPERTS
        ks, is_ = [], []
        for j in range(4):
            v = probs_v[pl.ds(off + j * _L, _L)]
            sk, si = plsc.sort_key_val(v, iota + j * _L, descending=True)
            ks.append(sk)
            is_.append(si)
        ak = _merge8(ks[0], ks[1], iota)
        ai = _merge8(is_[0], is_[1], iota)
        ak, ai = plsc.sort_key_val(ak, ai, descending=True)
        bk = _merge8(ks[2], ks[3], iota)
        bi = _merge8(is_[2], is_[3], iota)
        bk, bi = plsc.sort_key_val(bk, bi, descending=True)
        ck = _merge8(ak, bk, iota)
        ci = _merge8(ai, bi, iota)
        ck, ci = plsc.sort_key_val(ck, ci, descending=True)
        s8 = jnp.sum(jnp.where(iota < _TOP_K, ck, 0.0))
        s8v = jnp.maximum(jnp.broadcast_to(s8, (_L,)), 1e-6)
        return ck / s8v, ci

    def pair_body(i, carry):
        tw0, ti0 = one_row(2 * i)
        tw1, ti1 = one_row(2 * i + 1)
        tw = _merge8(tw0, tw1, iota)
        ti = _merge8(ti0, ti1, iota)
        topw_v[pl.ds(i * _L, _L)] = tw
        topi_v[pl.ds(i * _L, _L)] = ti
        return carry

    lax.fori_loop(0, rows // 2, pair_body, 0)

    obase = wid * rows * _TOP_K
    pltpu.sync_copy(topw_v, topw_hbm.at[pl.ds(obase, rows * _TOP_K)])
    pltpu.sync_copy(topi_v, topi_hbm.at[pl.ds(obase, rows * _TOP_K)])


@functools.partial(jax.jit, static_argnames=())
def kernel(x, prediction_error_ema, usage_penalty, alive_mask, gate_w, gate_b):
    tokens, feat = x.shape
    n_exp = gate_w.shape[0]
    grid = (tokens // _BLOCK,)

    wt = gate_w.T
    bias = (gate_b - usage_penalty).reshape(1, n_exp)
    nb = (1.0 - usage_penalty).reshape(1, n_exp)
    dead = (~alive_mask).astype(jnp.int32).reshape(1, n_exp)
    pe2d = prediction_error_ema.reshape(tokens, 1)

    out_shapes = (
        jax.ShapeDtypeStruct((tokens, n_exp), jnp.float32),
        jax.ShapeDtypeStruct((tokens, n_exp), jnp.float32),
    )
    in_specs = [
        pl.BlockSpec((_BLOCK, feat), lambda i: (i, 0)),
        pl.BlockSpec((_BLOCK, 1), lambda i: (i, 0)),
        pl.BlockSpec((feat, n_exp), lambda i: (0, 0)),
        pl.BlockSpec((1, n_exp), lambda i: (0, 0)),
        pl.BlockSpec((1, n_exp), lambda i: (0, 0)),
        pl.BlockSpec((1, n_exp), lambda i: (0, 0)),
    ]
    out_specs = (
        pl.BlockSpec((_BLOCK, n_exp), lambda i: (i, 0)),
        pl.BlockSpec((_BLOCK, n_exp), lambda i: (i, 0)),
    )
    n_chunks = 2
    ctok = tokens // n_chunks
    crows = ctok // _NW
    mesh = plsc.VectorSubcoreMesh(
        core_axis_name="c", subcore_axis_name="s",
        num_cores=_NC, num_subcores=_NS)
    sc_call = pl.kernel(
        _sc_topk,
        out_type=(
            jax.ShapeDtypeStruct((ctok * _TOP_K,), jnp.float32),
            jax.ShapeDtypeStruct((ctok * _TOP_K,), jnp.int32),
        ),
        mesh=mesh,
        compiler_params=pltpu.CompilerParams(needs_layout_passes=False),
        scratch_types=[
            pltpu.VMEM((crows * _NUM_EXPERTS,), jnp.float32),
            pltpu.VMEM((crows * _TOP_K,), jnp.float32),
            pltpu.VMEM((crows * _TOP_K,), jnp.int32),
        ],
    )

    logits_c, probs_c, topw_c, topi_c = [], [], [], []
    for c in range(n_chunks):
        coff = c * (ctok // _BLOCK)
        chunk_in_specs = [
            pl.BlockSpec((_BLOCK, feat), lambda i, o=coff: (i + o, 0)),
            pl.BlockSpec((_BLOCK, 1), lambda i, o=coff: (i + o, 0)),
            pl.BlockSpec((feat, n_exp), lambda i: (0, 0)),
            pl.BlockSpec((1, n_exp), lambda i: (0, 0)),
            pl.BlockSpec((1, n_exp), lambda i: (0, 0)),
            pl.BlockSpec((1, n_exp), lambda i: (0, 0)),
        ]
        lg, pr = pl.pallas_call(
            _gate_body,
            grid=(ctok // _BLOCK,),
            in_specs=chunk_in_specs,
            out_specs=out_specs,
            out_shape=(
                jax.ShapeDtypeStruct((ctok, n_exp), jnp.float32),
                jax.ShapeDtypeStruct((ctok, n_exp), jnp.float32),
            ),
            compiler_params=pltpu.CompilerParams(
                dimension_semantics=("parallel",)),
        )(x, pe2d, wt, bias, nb, dead)
        tw, ti = sc_call(pr.reshape(-1))
        logits_c.append(lg)
        probs_c.append(pr)
        topw_c.append(tw.reshape(ctok, _TOP_K))
        topi_c.append(ti.reshape(ctok, _TOP_K))
    logits = jnp.concatenate(logits_c, axis=0)
    probs = jnp.concatenate(probs_c, axis=0)
    topw = jnp.concatenate(topw_c, axis=0)
    topi = jnp.concatenate(topi_c, axis=0)
    return logits, probs, topw, topi


# speedup vs baseline: 1.4172x; 1.4172x over previous
import functools

import jax
import jax.numpy as jnp
from jax.experimental import pallas as pl
from jax.experimental.pallas import tpu as pltpu

_NUM_EXPERTS = 64
_TOP_K = 8
_BLOCK = 2048
_FMIN = float(jnp.finfo(jnp.float32).min)


def _routing_body(x_ref, pe_ref, wt_ref, bias_ref, nb_ref, dead_ref,
                  logits_ref, probs_ref, topw_ref, topi_ref):
    x = x_ref[...]
    logits = jax.lax.dot_general(
        x, wt_ref[...], (((1,), (0,)), ((), ())),
        preferred_element_type=jnp.float32)
    pe = pe_ref[...]
    logits = logits + bias_ref[...] + pe * nb_ref[...]
    logits = jnp.where(dead_ref[...] != 0, _FMIN, logits)
    logits_ref[...] = logits

    e = jnp.exp(logits)
    s = jnp.sum(e, axis=-1, keepdims=True)
    probs = e * (1.0 / s)
    probs_ref[...] = probs

    lane_f = jax.lax.broadcasted_iota(jnp.int32, probs.shape, 1).astype(
        jnp.float32)
    work = probs
    vals = []
    idxs_f = []
    for _ in range(_TOP_K):
        mx = jnp.max(work, axis=-1, keepdims=True)
        hit = work == mx
        idxf = jnp.min(jnp.where(hit, lane_f, float(_NUM_EXPERTS)), axis=-1,
                       keepdims=True)
        vals.append(mx)
        idxs_f.append(idxf)
        work = jnp.where(hit, -1.0, work)
    topw = jnp.concatenate(vals, axis=1)
    topi = jnp.concatenate(idxs_f, axis=1).astype(jnp.int32)
    denom = jnp.clip(jnp.sum(topw, axis=-1, keepdims=True), 1e-6, None)
    topw_ref[...] = topw * (1.0 / denom)
    topi_ref[...] = topi


@functools.partial(jax.jit, static_argnames=())
def kernel(x, prediction_error_ema, usage_penalty, alive_mask, gate_w, gate_b):
    tokens, feat = x.shape
    n_exp = gate_w.shape[0]
    grid = (tokens // _BLOCK,)

    wt = gate_w.T
    bias = (gate_b - usage_penalty).reshape(1, n_exp)
    nb = (1.0 - usage_penalty).reshape(1, n_exp)
    dead = (~alive_mask).astype(jnp.int32).reshape(1, n_exp)
    pe2d = prediction_error_ema.reshape(tokens, 1)

    out_shapes = (
        jax.ShapeDtypeStruct((tokens, n_exp), jnp.float32),
        jax.ShapeDtypeStruct((tokens, n_exp), jnp.float32),
        jax.ShapeDtypeStruct((tokens, _TOP_K), jnp.float32),
        jax.ShapeDtypeStruct((tokens, _TOP_K), jnp.int32),
    )
    in_specs = [
        pl.BlockSpec((_BLOCK, feat), lambda i: (i, 0)),
        pl.BlockSpec((_BLOCK, 1), lambda i: (i, 0)),
        pl.BlockSpec((feat, n_exp), lambda i: (0, 0)),
        pl.BlockSpec((1, n_exp), lambda i: (0, 0)),
        pl.BlockSpec((1, n_exp), lambda i: (0, 0)),
        pl.BlockSpec((1, n_exp), lambda i: (0, 0)),
    ]
    out_specs = (
        pl.BlockSpec((_BLOCK, n_exp), lambda i: (i, 0)),
        pl.BlockSpec((_BLOCK, n_exp), lambda i: (i, 0)),
        pl.BlockSpec((_BLOCK, _TOP_K), lambda i: (i, 0)),
        pl.BlockSpec((_BLOCK, _TOP_K), lambda i: (i, 0)),
    )
    return pl.pallas_call(
        _routing_body,
        grid=grid,
        in_specs=in_specs,
        out_specs=out_specs,
        out_shape=out_shapes,
        compiler_params=pltpu.CompilerParams(
            dimension_semantics=("parallel",)),
    )(x, pe2d, wt, bias, nb, dead)
